# baseline (speedup 1.0000x reference)
import jax
import jax.numpy as jnp
from jax.experimental import pallas as pl
from jax.experimental.pallas import tpu as pltpu

_NB = 4
_DEPTH = 4


def _compute_one(xb, tpT):
    hw, c = xb.shape
    mx = jnp.max(xb, axis=0)
    iota = jax.lax.broadcasted_iota(jnp.int32, (hw, c), 0)
    idx = jnp.min(jnp.where(xb == mx[None, :], iota, hw), axis=0)
    idx = jnp.where(mx == 0.0, hw - 1, idx)
    onehot = (iota == idx[None, :]).astype(jnp.float32)
    tmpl = jax.lax.dot_general(
        tpT, onehot, (((1,), (0,)), ((), ())),
        preferred_element_type=jnp.float32)
    return tmpl, jnp.maximum(xb * tmpl, 0.0)


def _body(x_hbm, tpT_ref, masked_hbm, tmpl_hbm,
          x_buf, m_buf, t_buf, in_sem, mo_sem, to_sem):
    nsteps = x_hbm.shape[0] // _NB
    tpT = tpT_ref[...]

    def in_copy(i, s):
        return pltpu.make_async_copy(
            x_hbm.at[pl.ds(i * _NB, _NB)], x_buf.at[s], in_sem.at[s])

    def m_copy(i, s):
        return pltpu.make_async_copy(
            m_buf.at[s], masked_hbm.at[pl.ds(i * _NB, _NB)], mo_sem.at[s])

    def t_copy(i, s):
        return pltpu.make_async_copy(
            t_buf.at[s], tmpl_hbm.at[pl.ds(i * _NB, _NB)], to_sem.at[s])

    for d in range(min(_DEPTH, nsteps)):
        in_copy(d, d).start()

    for i in range(nsteps):
        s = i % _DEPTH
        in_copy(i, s).wait()
        if i >= _DEPTH:
            m_copy(i - _DEPTH, s).wait()
            t_copy(i - _DEPTH, s).wait()
        for n in range(_NB):
            tmpl, masked = _compute_one(x_buf[s, n], tpT)
            t_buf[s, n] = tmpl
            m_buf[s, n] = masked
        m_copy(i, s).start()
        t_copy(i, s).start()
        if i + _DEPTH < nsteps:
            in_copy(i + _DEPTH, s).start()

    for i in range(max(0, nsteps - _DEPTH), nsteps):
        s = i % _DEPTH
        m_copy(i, s).wait()
        t_copy(i, s).wait()


def kernel(x, t_p):
    b, h, w, c = x.shape
    hw = h * w
    xr = jnp.reshape(x, (b, hw, c))
    tpT = jnp.transpose(jnp.reshape(t_p, (hw, hw)), (1, 0))
    masked_r, tmpl_r = pl.pallas_call(
        _body,
        in_specs=[
            pl.BlockSpec(memory_space=pl.ANY),
            pl.BlockSpec(memory_space=pltpu.VMEM),
        ],
        out_specs=[
            pl.BlockSpec(memory_space=pl.ANY),
            pl.BlockSpec(memory_space=pl.ANY),
        ],
        out_shape=[
            jax.ShapeDtypeStruct((b, hw, c), jnp.float32),
            jax.ShapeDtypeStruct((b, hw, c), jnp.float32),
        ],
        scratch_shapes=[
            pltpu.VMEM((_DEPTH, _NB, hw, c), jnp.float32),
            pltpu.VMEM((_DEPTH, _NB, hw, c), jnp.float32),
            pltpu.VMEM((_DEPTH, _NB, hw, c), jnp.float32),
            pltpu.SemaphoreType.DMA((_DEPTH,)),
            pltpu.SemaphoreType.DMA((_DEPTH,)),
            pltpu.SemaphoreType.DMA((_DEPTH,)),
        ],
    )(xr, tpT)
    masked = jnp.reshape(masked_r, (b, h, w, c))
    templates = jnp.reshape(tmpl_r, (b, h, w, c))
    return (masked, x, templates)
